# baseline (speedup 1.0000x reference)
import functools

import jax
import jax.numpy as jnp
from jax import lax
from jax.experimental import pallas as pl
from jax.experimental.pallas import tpu as pltpu
from jax.experimental.pallas import tpu_sc as plsc

_NUM_CORES = 2
_NUM_SUBCORES = 16
_NW = _NUM_CORES * _NUM_SUBCORES

_D = 1000
_DP = 1024


def _sc_gather_padded(idx3, table_p, *, batch, seq_p):
    rows_per_w = batch * seq_p // _NW
    chunk = 32
    n_chunks = rows_per_w // chunk

    mesh = plsc.VectorSubcoreMesh(
        core_axis_name="c",
        subcore_axis_name="s",
        num_cores=_NUM_CORES,
        num_subcores=_NUM_SUBCORES,
    )

    @functools.partial(
        pl.kernel,
        out_type=jax.ShapeDtypeStruct((batch * seq_p, _DP), jnp.float32),
        mesh=mesh,
        scratch_types=[
            pltpu.VMEM((n_chunks, chunk), jnp.int32),
            pltpu.VMEM((2, chunk, _DP), jnp.float32),
            pltpu.SemaphoreType.DMA,
            pltpu.SemaphoreType.DMA,
        ],
    )
    def gather_kernel(table_hbm, idx_hbm, out_hbm, idx_v, buf, gsem, ssem):
        wid = lax.axis_index("s") * _NUM_CORES + lax.axis_index("c")
        base = wid * rows_per_w
        pltpu.sync_copy(idx_hbm.at[wid], idx_v)

        pltpu.make_async_copy(table_hbm.at[idx_v.at[0]], buf.at[0], gsem).start()

        @pl.loop(0, n_chunks)
        def _(g):
            slot = lax.rem(g, 2)
            nslot = lax.rem(g + 1, 2)

            @pl.when(g >= 1)
            def _():
                pltpu.make_async_copy(
                    buf.at[nslot], out_hbm.at[pl.ds(0, chunk)], ssem
                ).wait()

            @pl.when(g + 1 < n_chunks)
            def _():
                pltpu.make_async_copy(
                    table_hbm.at[idx_v.at[g + 1]], buf.at[nslot], gsem
                ).start()

            pltpu.make_async_copy(
                table_hbm.at[idx_v.at[g]], buf.at[slot], gsem
            ).wait()
            pltpu.make_async_copy(
                buf.at[slot],
                out_hbm.at[pl.ds(base + g * chunk, chunk)],
                ssem,
            ).start()

        pltpu.make_async_copy(
            buf.at[lax.rem(n_chunks - 1, 2)], out_hbm.at[pl.ds(0, chunk)], ssem
        ).wait()

    return gather_kernel(table_p, idx3)


def _tc_relayout(padded2d, *, batch, seq, seq_p):
    blk = 8

    def copy_kernel(in_ref, out_ref):
        for i in range(blk):
            out_ref[i] = in_ref[pl.ds(i * seq_p, seq), :_D]

    return pl.pallas_call(
        copy_kernel,
        grid=(batch // blk,),
        in_specs=[
            pl.BlockSpec((blk * seq_p, _DP), lambda n: (n, 0)),
        ],
        out_specs=pl.BlockSpec((blk, seq, _D), lambda n: (n, 0, 0)),
        out_shape=jax.ShapeDtypeStruct((batch, seq, _D), jnp.float32),
        compiler_params=pltpu.CompilerParams(
            dimension_semantics=("arbitrary",),
        ),
    )(padded2d)


@functools.partial(jax.jit, static_argnames=("batch", "seq"))
def _embedding_lookup(idx, table, *, batch, seq):
    b_per_w = batch // _NW
    seq_p = ((seq + 7) // 8) * 8
    n_fill = seq_p - seq
    fill = (
        jnp.arange(_NW * b_per_w * n_fill, dtype=jnp.int32) % jnp.int32(1000)
    ).reshape(_NW, b_per_w, n_fill)
    idx3 = jnp.concatenate(
        [idx.reshape(_NW, b_per_w, seq).astype(jnp.int32), fill], axis=2
    ).reshape(_NW, b_per_w * seq_p // 32, 32)
    table_p = jnp.pad(table, ((0, 0), (0, _DP - _D)))

    padded = _sc_gather_padded(idx3, table_p, batch=batch, seq_p=seq_p)
    return _tc_relayout(padded, batch=batch, seq=seq, seq_p=seq_p)


def kernel(idx, token_embedding_table):
    B, T = idx.shape
    return _embedding_lookup(idx, token_embedding_table, batch=B, seq=T)


# speedup vs baseline: 1.4675x; 1.4675x over previous
import functools

import jax
import jax.numpy as jnp
from jax import lax
from jax.experimental import pallas as pl
from jax.experimental.pallas import tpu as pltpu
from jax.experimental.pallas import tpu_sc as plsc

_NUM_CORES = 2
_NUM_SUBCORES = 16
_NW = _NUM_CORES * _NUM_SUBCORES

_D = 1000
_DP = 1024
_DA = 896
_DT = _D - _DA


def _sc_gather(idx3, table_p, *, batch, seq, seq_a, seq_p):
    b_per_w = batch // _NW

    mesh = plsc.VectorSubcoreMesh(
        core_axis_name="c",
        subcore_axis_name="s",
        num_cores=_NUM_CORES,
        num_subcores=_NUM_SUBCORES,
    )

    @functools.partial(
        pl.kernel,
        out_type=jax.ShapeDtypeStruct((batch, seq, _D), jnp.float32),
        mesh=mesh,
        scratch_types=[
            pltpu.VMEM((b_per_w, seq), jnp.int32),
            pltpu.VMEM((2, seq, _DP), jnp.float32),
            pltpu.VMEM((seq_a, _DT), jnp.float32),
            [pltpu.VMEM((2, 128), jnp.float32) for _ in range(7)],
            pltpu.VMEM((2, _DT), jnp.float32),
            pltpu.SemaphoreType.DMA,
            pltpu.SemaphoreType.DMA,
        ],
    )
    def gather_kernel(
        table_hbm, idx_hbm, out_hbm, idx_v, buf, tail, t2, t2t, gsem, ssem
    ):
        wid = lax.axis_index("s") * _NUM_CORES + lax.axis_index("c")
        base = wid * b_per_w
        pltpu.sync_copy(idx_hbm.at[wid], idx_v)

        def drain_writebacks(s):
            pltpu.make_async_copy(
                buf.at[s].at[pl.ds(0, seq_a), pl.ds(0, _DA)],
                out_hbm.at[0].at[pl.ds(0, seq_a), pl.ds(0, _DA)],
                ssem,
            ).wait()
            pltpu.make_async_copy(
                tail,
                out_hbm.at[0].at[pl.ds(0, seq_a), pl.ds(_DA, _DT)],
                ssem,
            ).wait()
            for k in range(7):
                pltpu.make_async_copy(
                    t2[k],
                    out_hbm.at[0].at[pl.ds(seq_a, seq - seq_a), pl.ds(k * 128, 128)],
                    ssem,
                ).wait()
            pltpu.make_async_copy(
                t2t,
                out_hbm.at[0].at[pl.ds(seq_a, seq - seq_a), pl.ds(_DA, _DT)],
                ssem,
            ).wait()

        pltpu.make_async_copy(table_hbm.at[idx_v.at[0]], buf.at[0], gsem).start()

        @pl.loop(0, b_per_w)
        def _(g):
            slot = lax.rem(g, 2)
            nslot = lax.rem(g + 1, 2)

            @pl.when(g >= 1)
            def _():
                drain_writebacks(nslot)

            @pl.when(g + 1 < b_per_w)
            def _():
                pltpu.make_async_copy(
                    table_hbm.at[idx_v.at[g + 1]], buf.at[nslot], gsem
                ).start()

            pltpu.make_async_copy(
                table_hbm.at[idx_v.at[g]], buf.at[slot], gsem
            ).wait()

            @pl.loop(0, seq_a)
            def _(r):
                @pl.loop(0, 6, unroll=6)
                def _(k):
                    tail[r, pl.ds(k * 16, 16)] = buf[
                        slot, r, pl.ds(_DA + k * 16, 16)
                    ]

                tail[r, pl.ds(_DT - 16, 16)] = buf[
                    slot, r, pl.ds(_DA + _DT - 16, 16)
                ]

            b = base + g
            pltpu.make_async_copy(
                buf.at[slot].at[pl.ds(0, seq_a), pl.ds(0, _DA)],
                out_hbm.at[b].at[pl.ds(0, seq_a), pl.ds(0, _DA)],
                ssem,
            ).start()
            pltpu.make_async_copy(
                tail,
                out_hbm.at[b].at[pl.ds(0, seq_a), pl.ds(_DA, _DT)],
                ssem,
            ).start()
            for k in range(7):
                for j in range(2):
                    for m in range(8):
                        t2[k][j, pl.ds(m * 16, 16)] = buf[
                            slot, seq_a + j, pl.ds(k * 128 + m * 16, 16)
                        ]
            for j in range(2):
                for m in range(6):
                    t2t[j, pl.ds(m * 16, 16)] = buf[
                        slot, seq_a + j, pl.ds(_DA + m * 16, 16)
                    ]
                t2t[j, pl.ds(_DT - 16, 16)] = buf[
                    slot, seq_a + j, pl.ds(_DA + _DT - 16, 16)
                ]
            for k in range(7):
                pltpu.make_async_copy(
                    t2[k],
                    out_hbm.at[b].at[pl.ds(seq_a, seq - seq_a), pl.ds(k * 128, 128)],
                    ssem,
                ).start()
            pltpu.make_async_copy(
                t2t,
                out_hbm.at[b].at[pl.ds(seq_a, seq - seq_a), pl.ds(_DA, _DT)],
                ssem,
            ).start()

        drain_writebacks(lax.rem(b_per_w - 1, 2))

    return gather_kernel(table_p, idx3)


@functools.partial(jax.jit, static_argnames=("batch", "seq"))
def _embedding_lookup(idx, table, *, batch, seq):
    b_per_w = batch // _NW
    seq_a = (seq // 8) * 8
    seq_p = seq_a + 8
    idx3 = idx.reshape(_NW, b_per_w, seq).astype(jnp.int32)
    table_p = jnp.pad(table, ((0, 0), (0, _DP - _D)))

    return _sc_gather(
        idx3, table_p, batch=batch, seq=seq, seq_a=seq_a, seq_p=seq_p
    )


def kernel(idx, token_embedding_table):
    B, T = idx.shape
    return _embedding_lookup(idx, token_embedding_table, batch=B, seq=T)
